# baseline (speedup 1.0000x reference)
import functools

import jax
import jax.numpy as jnp
from jax import lax
from jax.experimental import pallas as pl
from jax.experimental.pallas import tpu as pltpu
from jax.experimental.pallas import tpu_sc as plsc

BATCH = 16384
D = 32
L = 16
NC = 2
NS = 16
NW = NC * NS
BPW = BATCH // NW
CHUNK = 128
NCHUNK = BPW // CHUNK


def _body(u_idx_hbm, i_idx_hbm, j_idx_hbm, ue_hbm, ie_hbm,
          out_ui_hbm, out_uj_hbm,
          idx_u, idx_i, idx_j, rows_u, rows_i, rows_j,
          out_ui_v, out_uj_v, sem_u, sem_i, sem_j):
    wid = lax.axis_index("s") * NC + lax.axis_index("c")
    base = wid * BPW

    pltpu.sync_copy(u_idx_hbm.at[pl.ds(wid * NCHUNK, NCHUNK)], idx_u)
    pltpu.sync_copy(i_idx_hbm.at[pl.ds(wid * NCHUNK, NCHUNK)], idx_i)
    pltpu.sync_copy(j_idx_hbm.at[pl.ds(wid * NCHUNK, NCHUNK)], idx_j)

    copies = []
    for k in range(NCHUNK):
        sl = pl.ds(k * CHUNK, CHUNK)
        copies.append(pltpu.async_copy(ue_hbm.at[idx_u.at[k]], rows_u.at[sl], sem_u))
        copies.append(pltpu.async_copy(ie_hbm.at[idx_i.at[k]], rows_i.at[sl], sem_i))
        copies.append(pltpu.async_copy(ie_hbm.at[idx_j.at[k]], rows_j.at[sl], sem_j))
    for c in copies:
        c.wait()

    lane = lax.iota(jnp.int32, L)

    def group(g, _):
        row0 = g * L
        ridx = row0 + lane
        acc_ui = jnp.zeros((L,), jnp.float32)
        acc_uj = jnp.zeros((L,), jnp.float32)
        for c in range(D):
            cidx = jnp.full((L,), c, jnp.int32)
            uc = plsc.load_gather(rows_u, [ridx, cidx])
            ic = plsc.load_gather(rows_i, [ridx, cidx])
            jc = plsc.load_gather(rows_j, [ridx, cidx])
            acc_ui = acc_ui + uc * ic
            acc_uj = acc_uj + uc * jc
        out_ui_v[pl.ds(row0, L)] = acc_ui
        out_uj_v[pl.ds(row0, L)] = acc_uj
        return 0

    lax.fori_loop(0, BPW // L, group, 0)

    pltpu.sync_copy(out_ui_v, out_ui_hbm.at[pl.ds(base, BPW)])
    pltpu.sync_copy(out_uj_v, out_uj_hbm.at[pl.ds(base, BPW)])


@jax.jit
def _bprmf(u_idx, i_idx, j_idx, user_emb, item_emb):
    mesh = plsc.VectorSubcoreMesh(core_axis_name="c", subcore_axis_name="s",
                                  num_cores=NC, num_subcores=NS)
    f = functools.partial(
        pl.kernel,
        out_type=[jax.ShapeDtypeStruct((BATCH,), jnp.float32),
                  jax.ShapeDtypeStruct((BATCH,), jnp.float32)],
        mesh=mesh,
        scratch_types=[
            pltpu.VMEM((NCHUNK, CHUNK), jnp.int32),
            pltpu.VMEM((NCHUNK, CHUNK), jnp.int32),
            pltpu.VMEM((NCHUNK, CHUNK), jnp.int32),
            pltpu.VMEM((BPW, D), jnp.float32),
            pltpu.VMEM((BPW, D), jnp.float32),
            pltpu.VMEM((BPW, D), jnp.float32),
            pltpu.VMEM((BPW,), jnp.float32),
            pltpu.VMEM((BPW,), jnp.float32),
            pltpu.SemaphoreType.DMA,
            pltpu.SemaphoreType.DMA,
            pltpu.SemaphoreType.DMA,
        ],
        compiler_params=pltpu.CompilerParams(needs_layout_passes=False,
                                             use_tc_tiling_on_sc=False),
    )(_body)
    out_ui, out_uj = f(u_idx, i_idx, j_idx, user_emb, item_emb)
    return (out_ui, out_uj)


def kernel(u_idx, i_idx, j_idx, user_emb, item_emb):
    u_idx = u_idx.astype(jnp.int32).reshape(NW * NCHUNK, CHUNK)
    i_idx = i_idx.astype(jnp.int32).reshape(NW * NCHUNK, CHUNK)
    j_idx = j_idx.astype(jnp.int32).reshape(NW * NCHUNK, CHUNK)
    return _bprmf(u_idx, i_idx, j_idx, user_emb, item_emb)


# speedup vs baseline: 2.3622x; 2.3622x over previous
import functools

import jax
import jax.numpy as jnp
from jax import lax
from jax.experimental import pallas as pl
from jax.experimental.pallas import tpu as pltpu
from jax.experimental.pallas import tpu_sc as plsc

NC, NS, L = 2, 16, 16
NW = NC * NS
BATCH = 16384
BPW = BATCH // NW
D = 32
V = 1000000
W = V // NW
CH = 512
NCHK = 62
WIN = CH * NCHK
ACC = 2048
STG = 3 * BATCH
DUMP = STG
SUB = 128
NSUB = BPW // SUB
VALID = 999936
TABMAX = 999424


def _body_a(ut, it, ut_tail, it_tail, u_idx, i_idx, j_idx, staging,
            qbuf, mylist, accb, stream, rowt, posid, tailv, cur,
            sem_s0, sem_s1, sem_w0, sem_w1):
    wid = lax.axis_index("s") * NC + lax.axis_index("c")
    c0 = wid * W
    start = (c0 >> 7) << 7
    lane = lax.iota(jnp.int32, L)
    sem_s = (sem_s0, sem_s1)
    sem_w = (sem_w0, sem_w1)

    def scan_array(arr_hbm, use):
        pltpu.sync_copy(arr_hbm, qbuf)

        def grp(g, _):
            r = qbuf[pl.ds(g * L, L)]
            m = (r >= c0) & (r < c0 + W) & (r < VALID)
            col = r - start
            pos = g * L + lane
            ent = (col << 16) | (use << 14) | pos
            pc = plsc.cumsum(m.astype(jnp.int32))
            cursor = cur[0]
            plsc.store_scatter(mylist, [cursor + pc - 1], ent, mask=m)
            cur[0] = cursor + pc[L - 1]
            return 0

        lax.fori_loop(0, BATCH // L, grp, 0)

    def drain_w(par):
        pltpu.make_async_copy(rowt.at[0], staging.at[pl.ds(DUMP, L)],
                              sem_w[par]).wait()

    def serve(gather_fn):
        n = cur[1]
        ng = (n + L - 1) >> 4
        npairs = (ng + 1) >> 1

        def subgrp(t, par):
            e = accb[pl.ds(t * L, L)]
            mv = (t * L + lane) < n
            dest = ((e >> 14) & 3) * BATCH + (e & (BATCH - 1))
            dest = jnp.where(mv, dest, DUMP + wid)
            parv = jnp.full((L,), par, jnp.int32)
            plsc.store_scatter(posid, [parv, lane], dest)
            for c in range(D):
                cv = jnp.full((L,), c, jnp.int32)
                vals = gather_fn(e, cv)
                plsc.store_scatter(rowt, [parv, lane, cv], vals)
            pltpu.async_copy(rowt.at[par], staging.at[posid.at[par]],
                             sem_w[par])

        def spair(p, _):
            @pl.when(p >= 1)
            def _():
                drain_w(0)
                drain_w(1)

            subgrp(2 * p, 0)

            @pl.when(2 * p + 1 < ng)
            def _():
                subgrp(2 * p + 1, 1)

            return 0

        lax.fori_loop(0, npairs, spair, 0)

        @pl.when(ng >= 1)
        def _():
            drain_w(0)

        @pl.when(jnp.logical_and(ng >= 2, (ng & 1) == 0))
        def _():
            drain_w(1)

        cur[1] = 0

    def run_pass(tab, arrays):
        cur[0] = 0
        cur[1] = 0
        for arr, use in arrays:
            scan_array(arr, use)

        def issue_chunk(k, par):
            off = pl.multiple_of(jnp.minimum(start + k * CH, TABMAX), 128)
            pltpu.async_copy(tab.at[:, pl.ds(off, CH)], stream.at[par],
                             sem_s[par])

        def wait_chunk(par):
            pltpu.make_async_copy(tab.at[:, pl.ds(0, CH)], stream.at[par],
                                  sem_s[par]).wait()

        def do_chunk(k, kpar):
            kv = jnp.full((L,), kpar, jnp.int32)

            def gf(e, cv):
                cin = (e >> 16) & (CH - 1)
                return plsc.load_gather(stream, [kv, cv, cin])

            nly = cur[0]
            ngs = (nly + L - 1) >> 4

            def mgrp(g, _):
                e = mylist[pl.ds(g * L, L)]
                mv = (g * L + lane) < nly
                ck = (e >> 16) >> 9
                m = mv & (ck == k)
                pc = plsc.cumsum(m.astype(jnp.int32))
                ac = cur[1]
                plsc.store_scatter(accb, [ac + pc - 1], e, mask=m)
                cur[1] = ac + pc[L - 1]

                @pl.when(cur[1] >= ACC - L)
                def _():
                    serve(gf)

                return 0

            lax.fori_loop(0, ngs, mgrp, 0)
            serve(gf)

        issue_chunk(0, 0)

        def cpair(p, _):
            issue_chunk(2 * p + 1, 1)
            wait_chunk(0)
            do_chunk(2 * p, 0)

            @pl.when(p < NCHK // 2 - 1)
            def _():
                issue_chunk(2 * p + 2, 0)

            wait_chunk(1)
            do_chunk(2 * p + 1, 1)
            return 0

        lax.fori_loop(0, NCHK // 2, cpair, 0)

    run_pass(ut, [(u_idx, 0)])
    run_pass(it, [(i_idx, 1), (j_idx, 2)])

    pltpu.sync_copy(ut_tail, tailv.at[0])
    pltpu.sync_copy(it_tail, tailv.at[1])
    cur[1] = 0

    def gf_tail(e, cv):
        cin = (e >> 16) & 127
        tsel = ((e >> 14) & 3 != 0).astype(jnp.int32)
        return plsc.load_gather(tailv, [tsel, cv, cin])

    for arr, use in ((u_idx, 0), (i_idx, 1), (j_idx, 2)):
        pltpu.sync_copy(arr, qbuf)

        def tgrp(g, _):
            gg = wid * (BPW // L) + g
            r = qbuf[pl.ds(gg * L, L)]
            m = r >= VALID
            pos = gg * L + lane
            ent = ((r - VALID) << 16) | (use << 14) | pos
            pc = plsc.cumsum(m.astype(jnp.int32))
            ac = cur[1]
            plsc.store_scatter(accb, [ac + pc - 1], ent, mask=m)
            cur[1] = ac + pc[L - 1]
            return 0

        lax.fori_loop(0, BPW // L, tgrp, 0)
    serve(gf_tail)


def _body_b(staging, out_ui, out_uj, su, si, sj, oui, ouj):
    wid = lax.axis_index("s") * NC + lax.axis_index("c")
    base = wid * BPW
    lane = lax.iota(jnp.int32, L)

    for sub in range(NSUB):
        sb = base + sub * SUB
        pltpu.sync_copy(staging.at[pl.ds(sb, SUB)], su)
        pltpu.sync_copy(staging.at[pl.ds(BATCH + sb, SUB)], si)
        pltpu.sync_copy(staging.at[pl.ds(2 * BATCH + sb, SUB)], sj)

        def group(g, _):
            ridx = g * L + lane
            acc_ui = jnp.zeros((L,), jnp.float32)
            acc_uj = jnp.zeros((L,), jnp.float32)
            for c in range(D):
                cv = jnp.full((L,), c, jnp.int32)
                uc = plsc.load_gather(su, [ridx, cv])
                ic = plsc.load_gather(si, [ridx, cv])
                jc = plsc.load_gather(sj, [ridx, cv])
                acc_ui = acc_ui + uc * ic
                acc_uj = acc_uj + uc * jc
            oui[pl.ds(sub * SUB + g * L, L)] = acc_ui
            ouj[pl.ds(sub * SUB + g * L, L)] = acc_uj
            return 0

        lax.fori_loop(0, SUB // L, group, 0)

    pltpu.sync_copy(oui, out_ui.at[pl.ds(base, BPW)])
    pltpu.sync_copy(ouj, out_uj.at[pl.ds(base, BPW)])


@jax.jit
def _bprmf(u_idx, i_idx, j_idx, user_emb, item_emb):
    mesh = plsc.VectorSubcoreMesh(core_axis_name="c", subcore_axis_name="s",
                                  num_cores=NC, num_subcores=NS)
    fa = functools.partial(
        pl.kernel,
        out_type=jax.ShapeDtypeStruct((STG + NW, 128), jnp.float32),
        mesh=mesh,
        scratch_types=[
            pltpu.VMEM((BATCH,), jnp.int32),
            pltpu.VMEM((STG + L,), jnp.int32),
            pltpu.VMEM((ACC,), jnp.int32),
            pltpu.VMEM((2, D, CH), jnp.float32),
            pltpu.VMEM((2, L, 128), jnp.float32),
            pltpu.VMEM((2, L), jnp.int32),
            pltpu.VMEM((2, D, 128), jnp.float32),
            pltpu.SMEM((8,), jnp.int32),
            pltpu.SemaphoreType.DMA,
            pltpu.SemaphoreType.DMA,
            pltpu.SemaphoreType.DMA,
            pltpu.SemaphoreType.DMA,
        ],
        compiler_params=pltpu.CompilerParams(needs_layout_passes=False),
    )(_body_a)

    fb = functools.partial(
        pl.kernel,
        out_type=[jax.ShapeDtypeStruct((BATCH,), jnp.float32),
                  jax.ShapeDtypeStruct((BATCH,), jnp.float32)],
        mesh=mesh,
        scratch_types=[
            pltpu.VMEM((SUB, 128), jnp.float32),
            pltpu.VMEM((SUB, 128), jnp.float32),
            pltpu.VMEM((SUB, 128), jnp.float32),
            pltpu.VMEM((BPW,), jnp.float32),
            pltpu.VMEM((BPW,), jnp.float32),
        ],
        compiler_params=pltpu.CompilerParams(needs_layout_passes=False),
    )(_body_b)

    ut_tail = jnp.pad(user_emb[VALID:].T, ((0, 0), (0, 128 - (V - VALID))))
    it_tail = jnp.pad(item_emb[VALID:].T, ((0, 0), (0, 128 - (V - VALID))))
    staging = fa(user_emb.T, item_emb.T, ut_tail, it_tail,
                 u_idx, i_idx, j_idx)
    out_ui, out_uj = fb(staging)
    return (out_ui, out_uj)


def kernel(u_idx, i_idx, j_idx, user_emb, item_emb):
    return _bprmf(u_idx.astype(jnp.int32), i_idx.astype(jnp.int32),
                  j_idx.astype(jnp.int32), user_emb, item_emb)


# speedup vs baseline: 2.7744x; 1.1745x over previous
import functools

import jax
import jax.numpy as jnp
from jax import lax
from jax.experimental import pallas as pl
from jax.experimental.pallas import tpu as pltpu
from jax.experimental.pallas import tpu_sc as plsc

NC, NS, L = 2, 16, 16
NW = NC * NS
BATCH = 16384
BPW = BATCH // NW
D = 32
V = 1000000
W = V // NW
CH = 512
NCHK = 62
WIN = CH * NCHK
ACC = 2048
STG = 3 * BATCH
DUMP = STG
SUB = 128
NSUB = BPW // SUB
VALID = 999936
QW_PAD = 64
TABMAX = 999424


def _body_a(ut, it, ut_tail, it_tail, u_idx, i_idx, j_idx, staging,
            qbuf, mylist, accb, stream, rowt, posid, tailv, cur,
            sem_s0, sem_s1, sem_w0, sem_w1):
    wid = lax.axis_index("s") * NC + lax.axis_index("c")
    c0 = wid * W
    start = (c0 >> 7) << 7
    lane = lax.iota(jnp.int32, L)
    sem_s = (sem_s0, sem_s1)
    sem_w = (sem_w0, sem_w1)

    QW = 4

    def scan_array(arr_hbm, use):
        pltpu.sync_copy(arr_hbm, qbuf)

        def grp(g, _):
            cursor = cur[0]
            tot = 0
            for q in range(QW):
                r = qbuf[pl.ds(g * (QW * L) + q * L, L)]
                m = (r >= c0) & (r < c0 + W) & (r < VALID)
                col = r - start
                pos = g * (QW * L) + q * L + lane
                ent = (col << 16) | (use << 14) | pos
                pc = plsc.cumsum(m.astype(jnp.int32))
                plsc.store_scatter(mylist, [cursor + tot + pc - 1], ent,
                                   mask=m)
                tot = tot + pc[L - 1]
            cur[0] = cursor + tot
            return 0

        lax.fori_loop(0, BATCH // (QW * L), grp, 0)

    def drain_w(par):
        pltpu.make_async_copy(rowt.at[0], staging.at[pl.ds(DUMP, L)],
                              sem_w[par]).wait()

    def serve(gather_fn):
        n = cur[1]
        ng = (n + L - 1) >> 4
        npairs = (ng + 1) >> 1

        def subgrp(t, par):
            e = accb[pl.ds(t * L, L)]
            mv = (t * L + lane) < n
            dest = ((e >> 14) & 3) * BATCH + (e & (BATCH - 1))
            dest = jnp.where(mv, dest, DUMP + wid)
            parv = jnp.full((L,), par, jnp.int32)
            plsc.store_scatter(posid, [parv, lane], dest)
            for c in range(D):
                cv = jnp.full((L,), c, jnp.int32)
                vals = gather_fn(e, cv)
                plsc.store_scatter(rowt, [parv, lane, cv], vals)
            pltpu.async_copy(rowt.at[par], staging.at[posid.at[par]],
                             sem_w[par])

        def spair(p, _):
            @pl.when(p >= 1)
            def _():
                drain_w(0)
                drain_w(1)

            subgrp(2 * p, 0)

            @pl.when(2 * p + 1 < ng)
            def _():
                subgrp(2 * p + 1, 1)

            return 0

        lax.fori_loop(0, npairs, spair, 0)

        @pl.when(ng >= 1)
        def _():
            drain_w(0)

        @pl.when(jnp.logical_and(ng >= 2, (ng & 1) == 0))
        def _():
            drain_w(1)

        cur[1] = 0

    def run_pass(tab, arrays):
        cur[0] = 0
        cur[1] = 0
        for arr, use in arrays:
            scan_array(arr, use)

        def issue_chunk(k, par):
            off = pl.multiple_of(jnp.minimum(start + k * CH, TABMAX), 128)
            pltpu.async_copy(tab.at[:, pl.ds(off, CH)], stream.at[par],
                             sem_s[par])

        def wait_chunk(par):
            pltpu.make_async_copy(tab.at[:, pl.ds(0, CH)], stream.at[par],
                                  sem_s[par]).wait()

        def do_chunk(k, kpar):
            kv = jnp.full((L,), kpar, jnp.int32)

            def gf(e, cv):
                cin = (e >> 16) & (CH - 1)
                return plsc.load_gather(stream, [kv, cv, cin])

            nly = cur[0]
            ngs = (nly + QW * L - 1) >> 6

            def mgrp(g, _):
                ac = cur[1]
                tot = 0
                for q in range(QW):
                    e = mylist[pl.ds(g * (QW * L) + q * L, L)]
                    mv = (g * (QW * L) + q * L + lane) < nly
                    ck = (e >> 16) >> 9
                    m = mv & (ck == k)
                    pc = plsc.cumsum(m.astype(jnp.int32))
                    plsc.store_scatter(accb, [ac + tot + pc - 1], e, mask=m)
                    tot = tot + pc[L - 1]
                cur[1] = ac + tot

                @pl.when(cur[1] >= ACC - QW * L)
                def _():
                    serve(gf)

                return 0

            lax.fori_loop(0, ngs, mgrp, 0)
            serve(gf)

        issue_chunk(0, 0)

        def cpair(p, _):
            issue_chunk(2 * p + 1, 1)
            wait_chunk(0)
            do_chunk(2 * p, 0)

            @pl.when(p < NCHK // 2 - 1)
            def _():
                issue_chunk(2 * p + 2, 0)

            wait_chunk(1)
            do_chunk(2 * p + 1, 1)
            return 0

        lax.fori_loop(0, NCHK // 2, cpair, 0)

    run_pass(ut, [(u_idx, 0)])
    run_pass(it, [(i_idx, 1), (j_idx, 2)])

    pltpu.sync_copy(ut_tail, tailv.at[0])
    pltpu.sync_copy(it_tail, tailv.at[1])
    cur[1] = 0

    def gf_tail(e, cv):
        cin = (e >> 16) & 127
        tsel = ((e >> 14) & 3 != 0).astype(jnp.int32)
        return plsc.load_gather(tailv, [tsel, cv, cin])

    for arr, use in ((u_idx, 0), (i_idx, 1), (j_idx, 2)):
        pltpu.sync_copy(arr, qbuf)

        def tgrp(g, _):
            gg = wid * (BPW // L) + g
            r = qbuf[pl.ds(gg * L, L)]
            m = r >= VALID
            pos = gg * L + lane
            ent = ((r - VALID) << 16) | (use << 14) | pos
            pc = plsc.cumsum(m.astype(jnp.int32))
            ac = cur[1]
            plsc.store_scatter(accb, [ac + pc - 1], ent, mask=m)
            cur[1] = ac + pc[L - 1]
            return 0

        lax.fori_loop(0, BPW // L, tgrp, 0)
    serve(gf_tail)


def _body_b(staging, out_ui, out_uj, su, si, sj, oui, ouj, semb0, semb1):
    wid = lax.axis_index("s") * NC + lax.axis_index("c")
    base = wid * BPW
    lane = lax.iota(jnp.int32, L)
    semb = (semb0, semb1)

    def issue(sub):
        par = sub & 1
        sb = base + sub * SUB
        return [
            pltpu.async_copy(staging.at[pl.ds(sb, SUB)], su.at[par], semb[par]),
            pltpu.async_copy(staging.at[pl.ds(BATCH + sb, SUB)], si.at[par],
                             semb[par]),
            pltpu.async_copy(staging.at[pl.ds(2 * BATCH + sb, SUB)],
                             sj.at[par], semb[par]),
        ]

    pend = issue(0)
    for sub in range(NSUB):
        par = sub & 1
        nxt = issue(sub + 1) if sub + 1 < NSUB else None
        for cp in pend:
            cp.wait()
        parv = jnp.full((L,), par, jnp.int32)

        def group(g, _):
            ridx = g * L + lane
            acc_ui = jnp.zeros((L,), jnp.float32)
            acc_uj = jnp.zeros((L,), jnp.float32)
            for c in range(D):
                cv = jnp.full((L,), c, jnp.int32)
                uc = plsc.load_gather(su, [parv, ridx, cv])
                ic = plsc.load_gather(si, [parv, ridx, cv])
                jc = plsc.load_gather(sj, [parv, ridx, cv])
                acc_ui = acc_ui + uc * ic
                acc_uj = acc_uj + uc * jc
            oui[pl.ds(sub * SUB + g * L, L)] = acc_ui
            ouj[pl.ds(sub * SUB + g * L, L)] = acc_uj
            return 0

        lax.fori_loop(0, SUB // L, group, 0)
        pend = nxt

    pltpu.sync_copy(oui, out_ui.at[pl.ds(base, BPW)])
    pltpu.sync_copy(ouj, out_uj.at[pl.ds(base, BPW)])


@jax.jit
def _bprmf(u_idx, i_idx, j_idx, user_emb, item_emb):
    mesh = plsc.VectorSubcoreMesh(core_axis_name="c", subcore_axis_name="s",
                                  num_cores=NC, num_subcores=NS)
    fa = functools.partial(
        pl.kernel,
        out_type=jax.ShapeDtypeStruct((STG + NW, 128), jnp.float32),
        mesh=mesh,
        scratch_types=[
            pltpu.VMEM((BATCH,), jnp.int32),
            pltpu.VMEM((STG + QW_PAD,), jnp.int32),
            pltpu.VMEM((ACC,), jnp.int32),
            pltpu.VMEM((2, D, CH), jnp.float32),
            pltpu.VMEM((2, L, 128), jnp.float32),
            pltpu.VMEM((2, L), jnp.int32),
            pltpu.VMEM((2, D, 128), jnp.float32),
            pltpu.SMEM((8,), jnp.int32),
            pltpu.SemaphoreType.DMA,
            pltpu.SemaphoreType.DMA,
            pltpu.SemaphoreType.DMA,
            pltpu.SemaphoreType.DMA,
        ],
        compiler_params=pltpu.CompilerParams(needs_layout_passes=False),
    )(_body_a)

    fb = functools.partial(
        pl.kernel,
        out_type=[jax.ShapeDtypeStruct((BATCH,), jnp.float32),
                  jax.ShapeDtypeStruct((BATCH,), jnp.float32)],
        mesh=mesh,
        scratch_types=[
            pltpu.VMEM((2, SUB, 128), jnp.float32),
            pltpu.VMEM((2, SUB, 128), jnp.float32),
            pltpu.VMEM((2, SUB, 128), jnp.float32),
            pltpu.VMEM((BPW,), jnp.float32),
            pltpu.VMEM((BPW,), jnp.float32),
            pltpu.SemaphoreType.DMA,
            pltpu.SemaphoreType.DMA,
        ],
        compiler_params=pltpu.CompilerParams(needs_layout_passes=False),
    )(_body_b)

    ut_tail = jnp.pad(user_emb[VALID:].T, ((0, 0), (0, 128 - (V - VALID))))
    it_tail = jnp.pad(item_emb[VALID:].T, ((0, 0), (0, 128 - (V - VALID))))
    staging = fa(user_emb.T, item_emb.T, ut_tail, it_tail,
                 u_idx, i_idx, j_idx)
    out_ui, out_uj = fb(staging)
    return (out_ui, out_uj)


def kernel(u_idx, i_idx, j_idx, user_emb, item_emb):
    return _bprmf(u_idx.astype(jnp.int32), i_idx.astype(jnp.int32),
                  j_idx.astype(jnp.int32), user_emb, item_emb)
